# Initial kernel scaffold; baseline (speedup 1.0000x reference)
import jax
import jax.numpy as jnp
from jax.experimental import pallas as pl


def kernel(x, edge_index, edge_attr, batch, params):
    raise NotImplementedError("write your pallas kernel here")


# speedup vs baseline: 1.3837x; 1.3837x over previous
import functools

import jax
import jax.numpy as jnp
from jax import lax
from jax.experimental import pallas as pl
from jax.experimental.pallas import tpu as pltpu
from jax.experimental.pallas import tpu_sc as plsc

N = 10000
E = 320000
G = 512
H = 64
NUM_LAYERS = 4
NUM_CLASSES = 3

BN = 2000
NB = N // BN
BE = 1600
NEB_PAD = (E + BE) // BE


_NTILES = 32
_EK = 80
_NP = 10240
_RPG = _NP // _NTILES
_ZR = 160
_DUMP = 10200
ESP = E + BE


def _sc_aggr_body(h_hbm, ef_hbm, src_hbm, dst_hbm, sp_hbm, le_hbm, cuts_hbm,
                  out_hbm, aggr, src_v, dst_v, rows_v, ef_v, stage_v, cuts_v,
                  sp_v, le_v, carry_v, sem):
    c = lax.axis_index("c")
    s = lax.axis_index("s")
    wid = c * 16 + s

    zero16 = jnp.zeros((16,), jnp.float32)

    def zrow(i, carry):
        for j in range(8):
            stage_v[i, pl.ds(j * 16, 16)] = zero16
        return carry
    lax.fori_loop(0, _ZR, zrow, 0)
    for j in range(8):
        carry_v[pl.ds(j * 16, 16)] = zero16

    for r in range(_RPG // _ZR):
        pltpu.sync_copy(stage_v, aggr.at[pl.ds(wid * _RPG + r * _ZR, _ZR)])

    pltpu.sync_copy(cuts_hbm, cuts_v)
    idx = jnp.full((16,), wid, jnp.int32)
    c_lo = jnp.max(plsc.load_gather(cuts_v, [idx]), axis=0)
    c_hi = jnp.max(plsc.load_gather(cuts_v, [idx + 1]), axis=0)
    base = jnp.bitwise_and(c_lo, -8)
    nch = (c_hi - base + (_EK - 1)) // _EK

    def chunk(t, carry):
        eb = pl.multiple_of(base + t * _EK, 8)
        pltpu.sync_copy(src_hbm.at[pl.ds(eb, _EK)], src_v)
        pltpu.sync_copy(dst_hbm.at[pl.ds(eb, _EK)], dst_v)
        pltpu.async_copy(h_hbm.at[src_v], rows_v, sem).wait()
        pltpu.sync_copy(ef_hbm.at[pl.ds(eb, _EK)], ef_v)
        pltpu.sync_copy(sp_hbm.at[pl.ds(eb, _EK)], sp_v)
        pltpu.sync_copy(le_hbm.at[pl.ds(eb, _EK)], le_v)

        cin = tuple(carry_v[pl.ds(j * 16, 16)] for j in range(8))

        def edge(e, pv):
            flagb = plsc.load_gather(sp_v, [jnp.full((16,), e, jnp.int32)]) == 1
            out = []
            for j in range(8):
                sl = pl.ds(j * 16, 16)
                m = jnp.maximum(rows_v[e, sl] + ef_v[e, sl], 0.0)
                m = m + jnp.where(flagb, pv[j], 0.0)
                rows_v[e, sl] = m
                out.append(m)
            return tuple(out)
        pv = lax.fori_loop(0, _EK, edge, cin)
        for j in range(8):
            carry_v[pl.ds(j * 16, 16)] = pv[j]

        for j in range(_EK // 16):
            sl = pl.ds(j * 16, 16)
            dv = dst_v[sl]
            own = (dv >= wid * _RPG) & (dv < (wid + 1) * _RPG) & (le_v[sl] == 1)
            dst_v[sl] = jnp.where(own, dv, _DUMP + wid)

        pltpu.sync_copy(rows_v, aggr.at[dst_v], add=True)
        return carry
    lax.fori_loop(0, nch, chunk, 0)

    for r in range(_RPG // _ZR):
        row0 = wid * _RPG + r * _ZR
        pltpu.sync_copy(aggr.at[pl.ds(row0, _ZR)], stage_v)
        pltpu.sync_copy(stage_v, out_hbm.at[pl.ds(row0, _ZR)])


@functools.lru_cache(maxsize=1)
def _get_sc_aggr():
    return functools.partial(
        pl.kernel,
        out_type=jax.ShapeDtypeStruct((_NP, 128), jnp.float32),
        mesh=plsc.VectorSubcoreMesh(core_axis_name="c", subcore_axis_name="s"),
        scratch_types=[
            pltpu.VMEM_SHARED((_NP, 128), jnp.float32),
            pltpu.VMEM((_EK,), jnp.int32),
            pltpu.VMEM((_EK,), jnp.int32),
            pltpu.VMEM((_EK, 128), jnp.float32),
            pltpu.VMEM((_EK, 128), jnp.float32),
            pltpu.VMEM((_ZR, 128), jnp.float32),
            pltpu.VMEM((48,), jnp.int32),
            pltpu.VMEM((_EK,), jnp.int32),
            pltpu.VMEM((_EK,), jnp.int32),
            pltpu.VMEM((128,), jnp.float32),
            pltpu.SemaphoreType.DMA,
        ],
        compiler_params=pltpu.CompilerParams(needs_layout_passes=False),
    )(_sc_aggr_body)


def _sc_aggr(h, ef, src, dst, sp, le, cuts):
    return _get_sc_aggr()(h, ef, src, dst, sp, le, cuts)


def _node_feat_body(x_ref, tabs_ref, out_ref):
    xb = x_ref[...]
    acc_l = None
    acc_r = None
    for c in range(8):
        oh = (xb[:, c][:, None] ==
              lax.broadcasted_iota(jnp.int32, (BN, 128), 1)).astype(jnp.float32)
        t = jnp.dot(oh, tabs_ref[c], preferred_element_type=jnp.float32, precision=lax.Precision.HIGHEST)
        if c == 0:
            acc_l = t
        elif acc_r is None:
            acc_r = t
        else:
            acc_r = acc_r + t
    out_ref[...] = jnp.concatenate([acc_l, acc_r], axis=1)


def _node_feat(x, node_tabs):
    return pl.pallas_call(
        _node_feat_body,
        grid=(NB,),
        in_specs=[
            pl.BlockSpec((BN, 8), lambda b: (b, 0)),
            pl.BlockSpec((8, 128, 64), lambda b: (0, 0, 0)),
        ],
        out_specs=pl.BlockSpec((BN, 128), lambda b: (b, 0)),
        out_shape=jax.ShapeDtypeStruct((N, 128), jnp.float32),
    )(x, node_tabs)


def _edge_feat_body(ea_ref, tabs_ref, out_ref):
    ea = ea_ref[...]
    halves = []
    for c in range(2):
        oh = (ea[:, c][:, None] ==
              lax.broadcasted_iota(jnp.int32, (BE, 32), 1)).astype(jnp.float32)
        halves.append(jnp.dot(oh, tabs_ref[c], preferred_element_type=jnp.float32, precision=lax.Precision.HIGHEST))
    out_ref[...] = jnp.concatenate(halves, axis=1)


def _edge_feat(edge_attr, edge_tabs):
    return pl.pallas_call(
        _edge_feat_body,
        grid=(NEB_PAD,),
        in_specs=[
            pl.BlockSpec((BE, 2), lambda b: (b, 0)),
            pl.BlockSpec((2, 32, 64), lambda b: (0, 0, 0)),
        ],
        out_specs=pl.BlockSpec((BE, 128), lambda b: (b, 0)),
        out_shape=jax.ShapeDtypeStruct((ESP, 128), jnp.float32),
    )(edge_attr, edge_tabs)


def _mm1_body(h_ref, a_ref, W1_ref, b1_ref, out_ref):
    z = h_ref[...] + a_ref[...]
    out_ref[...] = jnp.dot(z, W1_ref[...],
                           preferred_element_type=jnp.float32) + b1_ref[...]


def _mm1(h, aggr, W1, b1):
    full = lambda shape: pl.BlockSpec(shape, lambda b: tuple(0 for _ in shape))
    return pl.pallas_call(
        _mm1_body,
        grid=(NB,),
        in_specs=[
            pl.BlockSpec((BN, 128), lambda b: (b, 0)),
            pl.BlockSpec((BN, 128), lambda b: (b, 0)),
            full((128, 256)), full((1, 256)),
        ],
        out_specs=pl.BlockSpec((BN, 256), lambda b: (b, 0)),
        out_shape=jax.ShapeDtypeStruct((N, 256), jnp.float32),
    )(h, aggr, W1, b1)


def _mm2_body(y_ref, W2_ref, b2_ref, out_ref):
    out_ref[...] = jnp.dot(y_ref[...], W2_ref[...],
                           preferred_element_type=jnp.float32) + b2_ref[...]


def _mm2(y, W2, b2):
    full = lambda shape: pl.BlockSpec(shape, lambda b: tuple(0 for _ in shape))
    return pl.pallas_call(
        _mm2_body,
        grid=(NB,),
        in_specs=[
            pl.BlockSpec((BN, 256), lambda b: (b, 0)),
            full((256, 128)), full((1, 128)),
        ],
        out_specs=pl.BlockSpec((BN, 128), lambda b: (b, 0)),
        out_shape=jax.ShapeDtypeStruct((N, 128), jnp.float32),
    )(y, W2, b2)


def _batchnorm_x(h, g, b):
    mu = jnp.mean(h, axis=0)
    var = jnp.var(h, axis=0)
    return g * (h - mu) / jnp.sqrt(var + 1e-5) + b


def _iddot(y):
    eye = jnp.eye(y.shape[1], dtype=jnp.float32)
    return jnp.dot(y, eye, precision=lax.Precision.HIGHEST)


def _layer(final, h, aggr, W1, b1, g1, be1, W2, b2, g2, be2):
    y1 = _iddot(_mm1(h, aggr, W1, b1))
    zr = jax.nn.relu(_batchnorm_x(y1, g1, be1))
    y2 = _iddot(_mm2(zr, W2, b2))
    z2 = _batchnorm_x(y2, g2, be2)
    return z2 if final else jax.nn.relu(z2)


def _head_body(g_ref, W0_ref, b0_ref, W1_ref, b1_ref,
               W2_ref, b2_ref, W3_ref, b3_ref, out_ref):
    g = g_ref[...]
    g = jnp.maximum(jnp.dot(g, W0_ref[...], preferred_element_type=jnp.float32)
                    + b0_ref[...], 0.0)
    g = jnp.maximum(jnp.dot(g, W1_ref[...], preferred_element_type=jnp.float32)
                    + b1_ref[...], 0.0)
    g = jnp.maximum(jnp.dot(g, W2_ref[...], preferred_element_type=jnp.float32)
                    + b2_ref[...], 0.0)
    out_ref[...] = jnp.dot(g, W3_ref[...], preferred_element_type=jnp.float32) \
        + b3_ref[...]


def _head(g, W0, b0, W1, b1, W2, b2, W3p, b3p):
    full = lambda shape: pl.BlockSpec(shape, lambda: tuple(0 for _ in shape))
    return pl.pallas_call(
        _head_body,
        in_specs=[full((G, 128)),
                  full((128, 1024)), full((1, 1024)),
                  full((1024, 1024)), full((1, 1024)),
                  full((1024, 512)), full((1, 512)),
                  full((512, 128)), full((1, 128))],
        out_specs=full((G, 128)),
        out_shape=jax.ShapeDtypeStruct((G, 128), jnp.float32),
    )(g, W0, b0, W1, b1, W2, b2, W3p, b3p)


def _prep_tables(params):
    f32 = jnp.float32
    ar = jnp.arange(128, dtype=jnp.int32)
    bits128 = ((ar[:, None] >> jnp.arange(7, -1, -1, dtype=jnp.int32)) & 1).astype(f32)
    def r16(w):
        return lax.convert_element_type(
            lax.convert_element_type(w, jnp.bfloat16), jnp.float32)

    aW = r16(params["atom_lin_W"])
    ntabs = [jnp.pad(params["atom_emb"][:120], ((0, 8), (0, 0)))]
    for m in range(7):
        t = bits128 @ aW[8 * m:8 * m + 8]
        if m == 0:
            t = t + params["atom_lin_b"]
        ntabs.append(t)
    node_tabs = jnp.stack(ntabs)

    bits32 = bits128[:32]
    etab0 = jnp.pad(params["edge_emb"][:22], ((0, 10), (0, 0)))
    etab1 = bits32 @ r16(params["edge_lin_W"]) + params["edge_lin_b"]
    edge_tabs = jnp.stack([etab0, etab1])
    return node_tabs, edge_tabs


def _row(v):
    return v.reshape(1, -1)


def kernel(x, edge_index, edge_attr, batch, params):
    node_tabs, edge_tabs = _prep_tables(params)

    src = edge_index[0]
    dst = edge_index[1]
    order = jnp.argsort(dst, stable=True).astype(jnp.int32)
    dst_s = dst[order]
    src_s = src[order]
    ea_s = edge_attr[order]
    cuts = jnp.searchsorted(
        dst_s, jnp.arange(0, _NP + 1, _RPG, dtype=jnp.int32)).astype(jnp.int32)
    cuts = jnp.concatenate([cuts, jnp.full((15,), E, jnp.int32)])
    src_p = jnp.pad(src_s, (0, BE))
    dst_p = jnp.pad(dst_s, (0, BE), constant_values=_NP - 1)
    ea_p = jnp.pad(ea_s, ((0, BE), (0, 0)))
    sp = jnp.concatenate([jnp.zeros((1,), jnp.int32),
                          (dst_p[1:] == dst_p[:-1]).astype(jnp.int32)])
    le = jnp.concatenate([(dst_p[1:] != dst_p[:-1]).astype(jnp.int32),
                          jnp.ones((1,), jnp.int32)])

    h = _node_feat(x, node_tabs)
    ef = _edge_feat(ea_p, edge_tabs)

    for i in range(NUM_LAYERS):
        aggr = _sc_aggr(h, ef, src_p, dst_p, sp, le, cuts)
        h = _layer(
            i == NUM_LAYERS - 1, h, aggr[:N],
            params[f"conv{i}_W1"], _row(params[f"conv{i}_b1"]),
            _row(params[f"conv{i}_g1"]), _row(params[f"conv{i}_be1"]),
            params[f"conv{i}_W2"], _row(params[f"conv{i}_b2"]),
            _row(params[f"conv{i}_g2"]), _row(params[f"conv{i}_be2"]),
        )

    sums = jax.ops.segment_sum(h, batch, num_segments=G)
    cnt = jax.ops.segment_sum(jnp.ones((N, 1), dtype=jnp.float32), batch,
                              num_segments=G)
    g = sums / jnp.maximum(cnt, 1.0)
    W3p = jnp.pad(params["lin3_W"], ((0, 0), (0, 128 - NUM_CLASSES)))
    b3p = jnp.pad(params["lin3_b"], (0, 128 - NUM_CLASSES))
    out = _head(g, params["lin0_W"], _row(params["lin0_b"]),
                params["lin1_W"], _row(params["lin1_b"]),
                params["lin2_W"], _row(params["lin2_b"]),
                W3p, _row(b3p))
    return out[:, :NUM_CLASSES]
